# baseline (speedup 1.0000x reference)
import functools

import jax
import jax.numpy as jnp
from jax import lax
from jax.experimental import pallas as pl
from jax.experimental.pallas import tpu as pltpu
from jax.experimental.pallas import tpu_sc as plsc

B = 16384
NC, NS = 2, 16
NW = NC * NS
BPW = B // NW
L = 16
NF_G = 32
NF_M = 16
N_ROWS = 1000000


def _sc_gather_body(uids, iids, guf, gif, muf, mif,
                    gu_o, gi_o, mu_o, mi_o,
                    uix, iix, uidx, iidx, guv, giv, muv, miv, sem, osem):
    wid = lax.axis_index("s") * NC + lax.axis_index("c")
    base = wid * BPW
    pltpu.sync_copy(uids.at[pl.ds(base, BPW)], uix)
    pltpu.sync_copy(iids.at[pl.ds(base, BPW)], iix)

    def build(f, _):
        off = f * N_ROWS
        for j in range(BPW // L):
            sl = pl.ds(j * L, L)
            dsl = pl.ds(f * BPW + j * L, L)
            uidx[dsl] = uix[sl] + off
            iidx[dsl] = iix[sl] + off
        return _

    lax.fori_loop(0, NF_G, build, 0)

    copies = [
        pltpu.async_copy(guf.at[uidx], guv, sem),
        pltpu.async_copy(gif.at[iidx], giv, sem),
        pltpu.async_copy(muf.at[uidx.at[pl.ds(0, NF_M * BPW)]], muv, sem),
        pltpu.async_copy(mif.at[iidx.at[pl.ds(0, NF_M * BPW)]], miv, sem),
    ]
    for cp in copies:
        cp.wait()

    out_copies = []
    for f in range(NF_G):
        out_copies.append(pltpu.async_copy(
            guv.at[pl.ds(f * BPW, BPW)], gu_o.at[f, pl.ds(base, BPW)], osem))
        out_copies.append(pltpu.async_copy(
            giv.at[pl.ds(f * BPW, BPW)], gi_o.at[f, pl.ds(base, BPW)], osem))
    for f in range(NF_M):
        out_copies.append(pltpu.async_copy(
            muv.at[pl.ds(f * BPW, BPW)], mu_o.at[f, pl.ds(base, BPW)], osem))
        out_copies.append(pltpu.async_copy(
            miv.at[pl.ds(f * BPW, BPW)], mi_o.at[f, pl.ds(base, BPW)], osem))
    for cp in out_copies:
        cp.wait()


_sc_gather = functools.partial(
    pl.kernel,
    out_type=(
        jax.ShapeDtypeStruct((NF_G, B), jnp.float32),
        jax.ShapeDtypeStruct((NF_G, B), jnp.float32),
        jax.ShapeDtypeStruct((NF_M, B), jnp.float32),
        jax.ShapeDtypeStruct((NF_M, B), jnp.float32),
    ),
    mesh=plsc.VectorSubcoreMesh(core_axis_name="c", subcore_axis_name="s"),
    scratch_types=[
        pltpu.VMEM((BPW,), jnp.int32),
        pltpu.VMEM((BPW,), jnp.int32),
        pltpu.VMEM((NF_G * BPW,), jnp.int32),
        pltpu.VMEM((NF_G * BPW,), jnp.int32),
        pltpu.VMEM((NF_G * BPW,), jnp.float32),
        pltpu.VMEM((NF_G * BPW,), jnp.float32),
        pltpu.VMEM((NF_M * BPW,), jnp.float32),
        pltpu.VMEM((NF_M * BPW,), jnp.float32),
        pltpu.SemaphoreType.DMA,
        pltpu.SemaphoreType.DMA,
    ],
)(_sc_gather_body)


def _dense_body(gu, gi, mu, mi, w1at, w1bt, b1, w2t, b2, w3t, b3,
                wpg, wph, bp, out):
    h = jnp.dot(w1at[...], mu[...], preferred_element_type=jnp.float32)
    h += jnp.dot(w1bt[...], mi[...], preferred_element_type=jnp.float32)
    h = jnp.maximum(h + b1[...], 0.0)
    h = jnp.maximum(jnp.dot(w2t[...], h, preferred_element_type=jnp.float32) + b2[...], 0.0)
    h = jnp.maximum(jnp.dot(w3t[...], h, preferred_element_type=jnp.float32) + b3[...], 0.0)
    g = gu[...] * gi[...]
    logit = jnp.dot(wpg[...], g, preferred_element_type=jnp.float32)
    logit += jnp.dot(wph[...], h, preferred_element_type=jnp.float32)
    logit += bp[...]
    out[...] = 1.0 / (1.0 + jnp.exp(-logit))


def kernel(user_ids, item_ids, gmf_user_table, gmf_item_table,
           mlp_user_table, mlp_item_table, W1, b1, W2, b2, W3, b3, Wp, bp):
    gu_t, gi_t, mu_t, mi_t = _sc_gather(
        user_ids.astype(jnp.int32), item_ids.astype(jnp.int32),
        gmf_user_table.T.reshape(-1),
        gmf_item_table.T.reshape(-1),
        mlp_user_table.T.reshape(-1),
        mlp_item_table.T.reshape(-1),
    )
    w1at = W1[:16, :].T
    w1bt = W1[16:, :].T
    out_t = pl.pallas_call(
        _dense_body,
        out_shape=jax.ShapeDtypeStruct((1, B), jnp.float32),
    )(gu_t, gi_t, mu_t, mi_t, w1at, w1bt, b1.reshape(32, 1), W2.T,
      b2.reshape(16, 1), W3.T, b3.reshape(8, 1), Wp[:32, 0].reshape(1, 32),
      Wp[32:, 0].reshape(1, 8), bp.reshape(1, 1))
    return out_t.reshape(B, 1)
